# baseline (speedup 1.0000x reference)
import jax
import jax.numpy as jnp
from jax.experimental import pallas as pl
from jax.experimental.pallas import tpu as pltpu

B, N, D = 32, 576, 768
ROWS = B * N
IBLK = 6144
OBLK = 3072


def _body(idx_ref, embed_ref, tok_ref, out_ref):
    row = embed_ref[pl.ds(idx_ref[0], 1), :]
    i = pl.program_id(0)
    half = (i % 2) * OBLK
    out_ref[...] = tok_ref[pl.ds(half, OBLK), :] + row


def kernel(tokens, embed, frame_idx):
    idx = jnp.asarray(frame_idx, dtype=jnp.int32).reshape((1,))
    tok2 = tokens.reshape(ROWS, D)
    out = pl.pallas_call(
        _body,
        grid=(ROWS // OBLK,),
        in_specs=[
            pl.BlockSpec(memory_space=pltpu.MemorySpace.SMEM),
            pl.BlockSpec((embed.shape[0], D), lambda i: (0, 0)),
            pl.BlockSpec((IBLK, D), lambda i: (i // 2, 0)),
        ],
        out_specs=pl.BlockSpec((OBLK, D), lambda i: (i, 0)),
        out_shape=jax.ShapeDtypeStruct((ROWS, D), tokens.dtype),
        compiler_params=pltpu.CompilerParams(
            vmem_limit_bytes=60 * 1024 * 1024,
        ),
    )(idx, embed, tok2)
    return out.reshape(B, N, D)
